# baseline (speedup 1.0000x reference)
import functools

import jax
import jax.numpy as jnp
from jax import lax
from jax.experimental import pallas as pl
from jax.experimental.pallas import tpu as pltpu
from jax.experimental.pallas import tpu_sc as plsc

_NC, _NS = 2, 16
_NW = _NC * _NS


def _widen_kernel(V, D):
    chunk = 400
    per_w = (V // _NW) // (2 * chunk) * (2 * chunk)
    n2 = per_w // (2 * chunk)
    rem = (V - per_w * _NW) // chunk
    tail = V - rem * chunk
    mesh = plsc.VectorSubcoreMesh(core_axis_name="c", subcore_axis_name="s")

    @functools.partial(
        pl.kernel,
        mesh=mesh,
        out_type=jax.ShapeDtypeStruct((V, 2 * D), jnp.float32),
        scratch_types=[
            pltpu.VMEM((chunk, 2 * D), jnp.float32),
            pltpu.VMEM((chunk, 2 * D), jnp.float32),
            pltpu.SemaphoreType.DMA,
            pltpu.SemaphoreType.DMA,
        ],
        compiler_params=pltpu.CompilerParams(use_tc_tiling_on_sc=True),
    )
    def k(table_hbm, wide_hbm, buf0, buf1, sem_r, sem_w):
        w = lax.axis_index("s") * _NC + lax.axis_index("c")
        base = w * per_w

        def read(buf, off):
            return pltpu.async_copy(
                table_hbm.at[pl.ds(off, chunk)],
                buf.at[pl.ds(0, chunk), pl.ds(0, D)],
                sem_r,
            )

        def write(buf, off):
            return pltpu.async_copy(buf, wide_hbm.at[pl.ds(off, chunk)], sem_w)

        def body(j2, carry):
            off = base + j2 * (2 * chunk)
            r0 = read(buf0, off)
            r0.wait()
            w0 = write(buf0, off)
            r1 = read(buf1, off + chunk)
            r1.wait()
            w1 = write(buf1, off + chunk)
            w0.wait()
            w1.wait()
            return carry

        lax.fori_loop(0, n2, body, 0)

        if rem:

            @pl.when(w < rem)
            def _():
                off = tail + w * chunk
                read(buf0, off).wait()
                write(buf0, off).wait()

    return k


def _gather_kernel(R, S, V, D, rows_per_w, G):
    n2 = rows_per_w // (2 * G)
    mesh = plsc.VectorSubcoreMesh(core_axis_name="c", subcore_axis_name="s")

    @functools.partial(
        pl.kernel,
        mesh=mesh,
        out_type=jax.ShapeDtypeStruct((R, S, D), jnp.float32),
        scratch_types=[
            pltpu.VMEM((G, S), jnp.int32),
            pltpu.VMEM((G, S), jnp.int32),
            pltpu.VMEM((G, S, 2 * D), jnp.float32),
            pltpu.VMEM((G, S, 2 * D), jnp.float32),
            pltpu.SemaphoreType.DMA,
            pltpu.SemaphoreType.DMA,
        ],
        compiler_params=pltpu.CompilerParams(use_tc_tiling_on_sc=False),
    )
    def k(idx_hbm, table_hbm, out_hbm, ibufa, ibufb, bufa, bufb, sem_g, sem_w):
        wid = lax.axis_index("s") * _NC + lax.axis_index("c")
        base = wid * rows_per_w

        def stage(ibuf, r0):
            pltpu.sync_copy(idx_hbm.at[pl.ds(r0, G)], ibuf)

        def gathers(ibuf, buf):
            copies = [
                pltpu.async_copy(table_hbm.at[ibuf.at[j]], buf.at[j], sem_g)
                for j in range(G)
            ]
            for c in copies:
                c.wait()

        def write(buf, r0):
            return pltpu.async_copy(
                buf.at[pl.ds(0, G), pl.ds(0, S), pl.ds(0, D)],
                out_hbm.at[pl.ds(r0, G)],
                sem_w,
            )

        def body(k2, carry):
            r0 = base + k2 * (2 * G)
            r1 = r0 + G
            stage(ibufa, r0)
            stage(ibufb, r1)
            gathers(ibufa, bufa)
            wa = write(bufa, r0)
            gathers(ibufb, bufb)
            wb = write(bufb, r1)
            wa.wait()
            wb.wait()
            return carry

        lax.fori_loop(0, n2, body, 0)

    return k


def kernel(input, weight):
    R, S = input.shape
    V, D = weight.shape
    rows_per_w = R // _NW
    G = 8

    idx = input.astype(jnp.int32)
    wide = jax.lax.dot(
        weight,
        jnp.eye(D, 2 * D, dtype=jnp.float32),
        precision=jax.lax.Precision.HIGHEST,
    )
    return _gather_kernel(R, S, V, D, rows_per_w, G)(idx, wide)


# speedup vs baseline: 1.2533x; 1.2533x over previous
import functools

import jax
import jax.numpy as jnp
from jax import lax
from jax.experimental import pallas as pl
from jax.experimental.pallas import tpu as pltpu
from jax.experimental.pallas import tpu_sc as plsc

_NC, _NS = 2, 16
_NW = _NC * _NS


def _widen_kernel(V, D):
    chunk = 400
    per_w = (V // _NW) // (2 * chunk) * (2 * chunk)
    n2 = per_w // (2 * chunk)
    rem = (V - per_w * _NW) // chunk
    tail = V - rem * chunk
    mesh = plsc.VectorSubcoreMesh(core_axis_name="c", subcore_axis_name="s")

    @functools.partial(
        pl.kernel,
        mesh=mesh,
        out_type=jax.ShapeDtypeStruct((V, 2 * D), jnp.float32),
        scratch_types=[
            pltpu.VMEM((chunk, 2 * D), jnp.float32),
            pltpu.VMEM((chunk, 2 * D), jnp.float32),
            pltpu.SemaphoreType.DMA,
            pltpu.SemaphoreType.DMA,
        ],
        compiler_params=pltpu.CompilerParams(use_tc_tiling_on_sc=True),
    )
    def k(table_hbm, wide_hbm, buf0, buf1, sem_r, sem_w):
        w = lax.axis_index("s") * _NC + lax.axis_index("c")
        base = w * per_w

        def read(buf, off):
            return pltpu.async_copy(
                table_hbm.at[pl.ds(off, chunk)],
                buf.at[pl.ds(0, chunk), pl.ds(0, D)],
                sem_r,
            )

        def write(buf, off):
            return pltpu.async_copy(buf, wide_hbm.at[pl.ds(off, chunk)], sem_w)

        def body(j2, carry):
            off = base + j2 * (2 * chunk)
            r0 = read(buf0, off)
            r0.wait()
            w0 = write(buf0, off)
            r1 = read(buf1, off + chunk)
            r1.wait()
            w1 = write(buf1, off + chunk)
            w0.wait()
            w1.wait()
            return carry

        lax.fori_loop(0, n2, body, 0)

        if rem:

            @pl.when(w < rem)
            def _():
                off = tail + w * chunk
                read(buf0, off).wait()
                write(buf0, off).wait()

    return k


def _gather_kernel(R, S, V, D, rows_per_w, G):
    n2 = rows_per_w // (2 * G)
    mesh = plsc.VectorSubcoreMesh(core_axis_name="c", subcore_axis_name="s")

    @functools.partial(
        pl.kernel,
        mesh=mesh,
        out_type=jax.ShapeDtypeStruct((R, S, D), jnp.float32),
        scratch_types=[
            pltpu.VMEM((G, S), jnp.int32),
            pltpu.VMEM((G, S), jnp.int32),
            pltpu.VMEM((G, S, 2 * D), jnp.float32),
            pltpu.VMEM((G, S, 2 * D), jnp.float32),
            pltpu.SemaphoreType.DMA,
            pltpu.SemaphoreType.DMA,
        ],
        compiler_params=pltpu.CompilerParams(use_tc_tiling_on_sc=False),
    )
    def k(idx_hbm, table_hbm, out_hbm, ibufa, ibufb, bufa, bufb, sem_g, sem_w):
        wid = lax.axis_index("s") * _NC + lax.axis_index("c")
        base = wid * rows_per_w

        def stage(ibuf, r0):
            pltpu.sync_copy(idx_hbm.at[pl.ds(r0, G)], ibuf)

        def gathers(ibuf, buf):
            copies = [
                pltpu.async_copy(table_hbm.at[ibuf.at[j]], buf.at[j], sem_g)
                for j in range(G)
            ]
            for c in copies:
                c.wait()

        def write(buf, r0):
            return pltpu.async_copy(
                buf.at[pl.ds(0, G), pl.ds(0, S), pl.ds(0, D)],
                out_hbm.at[pl.ds(r0, G)],
                sem_w,
            )

        def body(k2, carry):
            r0 = base + k2 * (2 * G)
            r1 = r0 + G
            stage(ibufa, r0)
            stage(ibufb, r1)
            gathers(ibufa, bufa)
            wa = write(bufa, r0)
            gathers(ibufb, bufb)
            wb = write(bufb, r1)
            wa.wait()
            wb.wait()
            return carry

        lax.fori_loop(0, n2, body, 0)

    return k


def kernel(input, weight):
    R, S = input.shape
    V, D = weight.shape
    rows_per_w = R // _NW
    G = 8

    idx = input.astype(jnp.int32)
    wide = jax.lax.dot(
        weight,
        jnp.eye(D, 2 * D, dtype=jnp.float32),
        precision=jax.lax.Precision.HIGH,
    )
    return _gather_kernel(R, S, V, D, rows_per_w, G)(idx, wide)


# speedup vs baseline: 1.3671x; 1.0908x over previous
import functools

import jax
import jax.numpy as jnp
from jax import lax
from jax.experimental import pallas as pl
from jax.experimental.pallas import tpu as pltpu
from jax.experimental.pallas import tpu_sc as plsc

_NC, _NS = 2, 16
_NW = _NC * _NS


def _widen_kernel(V, D):
    chunk = 400
    per_w = (V // _NW) // (2 * chunk) * (2 * chunk)
    n2 = per_w // (2 * chunk)
    rem = (V - per_w * _NW) // chunk
    tail = V - rem * chunk
    mesh = plsc.VectorSubcoreMesh(core_axis_name="c", subcore_axis_name="s")

    @functools.partial(
        pl.kernel,
        mesh=mesh,
        out_type=jax.ShapeDtypeStruct((V, 2 * D), jnp.float32),
        scratch_types=[
            pltpu.VMEM((chunk, 2 * D), jnp.float32),
            pltpu.VMEM((chunk, 2 * D), jnp.float32),
            pltpu.SemaphoreType.DMA,
            pltpu.SemaphoreType.DMA,
        ],
        compiler_params=pltpu.CompilerParams(use_tc_tiling_on_sc=True),
    )
    def k(table_hbm, wide_hbm, buf0, buf1, sem_r, sem_w):
        w = lax.axis_index("s") * _NC + lax.axis_index("c")
        base = w * per_w

        def read(buf, off):
            return pltpu.async_copy(
                table_hbm.at[pl.ds(off, chunk)],
                buf.at[pl.ds(0, chunk), pl.ds(0, D)],
                sem_r,
            )

        def write(buf, off):
            return pltpu.async_copy(buf, wide_hbm.at[pl.ds(off, chunk)], sem_w)

        def body(j2, carry):
            off = base + j2 * (2 * chunk)
            r0 = read(buf0, off)
            r0.wait()
            w0 = write(buf0, off)
            r1 = read(buf1, off + chunk)
            r1.wait()
            w1 = write(buf1, off + chunk)
            w0.wait()
            w1.wait()
            return carry

        lax.fori_loop(0, n2, body, 0)

        if rem:

            @pl.when(w < rem)
            def _():
                off = tail + w * chunk
                read(buf0, off).wait()
                write(buf0, off).wait()

    return k


def _gather_kernel(R, S, V, D, rows_per_w, G):
    n2 = rows_per_w // (2 * G)
    mesh = plsc.VectorSubcoreMesh(core_axis_name="c", subcore_axis_name="s")

    @functools.partial(
        pl.kernel,
        mesh=mesh,
        out_type=jax.ShapeDtypeStruct((R, S, D), jnp.float32),
        scratch_types=[
            pltpu.VMEM((G, S), jnp.int32),
            pltpu.VMEM((G, S), jnp.int32),
            pltpu.VMEM((G, S, 2 * D), jnp.float32),
            pltpu.VMEM((G, S, 2 * D), jnp.float32),
            pltpu.SemaphoreType.DMA,
            pltpu.SemaphoreType.DMA,
        ],
        compiler_params=pltpu.CompilerParams(use_tc_tiling_on_sc=False),
    )
    def k(idx_hbm, table_hbm, out_hbm, ibufa, ibufb, bufa, bufb, sem_g, sem_w):
        wid = lax.axis_index("s") * _NC + lax.axis_index("c")
        base = wid * rows_per_w

        def stage(ibuf, r0):
            pltpu.sync_copy(idx_hbm.at[pl.ds(r0, G)], ibuf)

        def gathers(ibuf, buf):
            copies = [
                pltpu.async_copy(table_hbm.at[ibuf.at[j]], buf.at[j], sem_g)
                for j in range(G)
            ]
            for c in copies:
                c.wait()

        def write(buf, r0):
            return pltpu.async_copy(
                buf.at[pl.ds(0, G), pl.ds(0, S), pl.ds(0, D)],
                out_hbm.at[pl.ds(r0, G)],
                sem_w,
            )

        def body(k2, carry):
            r0 = base + k2 * (2 * G)
            r1 = r0 + G
            stage(ibufa, r0)
            stage(ibufb, r1)
            gathers(ibufa, bufa)
            wa = write(bufa, r0)
            gathers(ibufb, bufb)
            wb = write(bufb, r1)
            wa.wait()
            wb.wait()
            return carry

        lax.fori_loop(0, n2, body, 0)

    return k


def kernel(input, weight):
    R, S = input.shape
    V, D = weight.shape
    rows_per_w = R // _NW
    G = 8

    idx = input.astype(jnp.int32)
    wide = jax.lax.dot(
        weight,
        jnp.eye(D, 2 * D, dtype=jnp.float32),
        precision=jax.lax.Precision.DEFAULT,
    )
    return _gather_kernel(R, S, V, D, rows_per_w, G)(idx, wide)
